# Initial kernel scaffold; baseline (speedup 1.0000x reference)
import functools
import jax
import jax.numpy as jnp
from jax import lax
from jax.experimental import pallas as pl
from jax.experimental.pallas import tpu as pltpu
from jax.experimental.pallas import tpu_sc as plsc

N = 10000
E = 320000
D = 128

NC = 2
NS = 16
NW = NC * NS

CH = 128
EP_PER_W = 10112
EP = NW * EP_PER_W
NCHUNK = EP_PER_W // CH
N_PAD = 10240
ROWS_PER_TILE = N_PAD // NS
NDUMP = ROWS_PER_TILE // CH

_mesh = plsc.VectorSubcoreMesh(core_axis_name="c", subcore_axis_name="s")


@functools.partial(
    pl.kernel,
    out_type=jax.ShapeDtypeStruct((NC, N_PAD), jnp.float32),
    mesh=_mesh,
    scratch_types=[
        pltpu.MemoryRef((N_PAD,), jnp.float32, pltpu.VMEM_SHARED),
        pltpu.MemoryRef((CH,), jnp.int32, pltpu.VMEM),
        pltpu.MemoryRef((CH,), jnp.float32, pltpu.VMEM),
        pltpu.MemoryRef((ROWS_PER_TILE,), jnp.float32, pltpu.VMEM),
        pltpu.SemaphoreType.DMA,
    ],
)
def _sc_degree(dst_hbm, ones_hbm, zvec_hbm, deg_out, acc_sh, idx_v, ones_v,
               zbuf_v, sem):
    c = lax.axis_index("c")
    s = lax.axis_index("s")
    wid = c * NS + s
    pltpu.sync_copy(zvec_hbm, zbuf_v)
    pltpu.sync_copy(zbuf_v, acc_sh.at[pl.ds(s * ROWS_PER_TILE, ROWS_PER_TILE)])
    pltpu.sync_copy(ones_hbm, ones_v)
    plsc.subcore_barrier()

    def body(j, carry):
        base = wid * EP_PER_W + j * CH
        pltpu.sync_copy(dst_hbm.at[pl.ds(base, CH)], idx_v)
        pltpu.sync_copy(ones_v, acc_sh.at[idx_v], add=True)
        return carry

    lax.fori_loop(0, NCHUNK, body, 0)
    plsc.subcore_barrier()
    r0 = s * ROWS_PER_TILE
    pltpu.sync_copy(acc_sh.at[pl.ds(r0, ROWS_PER_TILE)], zbuf_v)
    pltpu.sync_copy(zbuf_v, deg_out.at[c, pl.ds(r0, ROWS_PER_TILE)])


@functools.partial(
    pl.kernel,
    out_type=jax.ShapeDtypeStruct((NC, N_PAD, D), jnp.float32),
    mesh=_mesh,
    scratch_types=[
        pltpu.MemoryRef((N_PAD, D), jnp.float32, pltpu.VMEM_SHARED),
        pltpu.MemoryRef((CH,), jnp.int32, pltpu.VMEM),
        pltpu.MemoryRef((CH,), jnp.int32, pltpu.VMEM),
        pltpu.MemoryRef((CH, D), jnp.float32, pltpu.VMEM),
        pltpu.SemaphoreType.DMA,
    ],
)
def _sc_aggregate(xws_hbm, src_hbm, dst_hbm, zrows_hbm, agg_out, acc_sh,
                  idxs_v, idxd_v, rows_v, sem):
    c = lax.axis_index("c")
    s = lax.axis_index("s")
    wid = c * NS + s
    pltpu.sync_copy(zrows_hbm, rows_v)
    for k in range(NDUMP):
        pltpu.sync_copy(
            rows_v, acc_sh.at[pl.ds(s * ROWS_PER_TILE + k * CH, CH), :])
    plsc.subcore_barrier()

    def body(j, carry):
        base = wid * EP_PER_W + j * CH
        pltpu.sync_copy(src_hbm.at[pl.ds(base, CH)], idxs_v)
        pltpu.sync_copy(dst_hbm.at[pl.ds(base, CH)], idxd_v)
        pltpu.async_copy(xws_hbm.at[idxs_v], rows_v, sem).wait()
        pltpu.sync_copy(rows_v, acc_sh.at[idxd_v], add=True)
        return carry

    lax.fori_loop(0, NCHUNK, body, 0)
    plsc.subcore_barrier()
    for k in range(NDUMP):
        r0 = s * ROWS_PER_TILE + k * CH
        pltpu.sync_copy(acc_sh.at[pl.ds(r0, CH), :], rows_v)
        pltpu.sync_copy(rows_v, agg_out.at[c, pl.ds(r0, CH), :])


def _tc_pre_body(degp_ref, t_ref, z_ref, wt1_ref, bt1_ref, wt2_ref, bt2_ref,
                 wenc_ref, xws1_ref, dinv_ref, temb_ref):
    deg = degp_ref[0, :] + degp_ref[1, :] + 1.0
    dinv_all = lax.rsqrt(deg)
    dinv = dinv_all[:N].reshape(N, 1)
    dinv_ref[...] = dinv
    t_in = t_ref[...].astype(jnp.float32)
    h = t_in * wt1_ref[...] + bt1_ref[...]
    h = jax.nn.gelu(h, approximate=False)
    temb_ref[...] = jnp.dot(h, wt2_ref[...],
                            preferred_element_type=jnp.float32) + bt2_ref[...]
    xw1 = jnp.dot(z_ref[...], wenc_ref[...],
                  preferred_element_type=jnp.float32)
    xws1_ref[:N, :] = xw1 * dinv
    xws1_ref[N:, :] = jnp.zeros((N_PAD - N, D), jnp.float32)


def _tc_mid_body(p_ref, xws1_ref, dinv_ref, temb_ref, benc_ref, wdec_ref,
                 xws2_ref):
    dinv = dinv_ref[...]
    agg = p_ref[0, :N, :] + p_ref[1, :N, :] + xws1_ref[:N, :]
    pre = dinv * agg + benc_ref[...] + temb_ref[...]
    h1 = jnp.where(pre > 0, pre, jnp.expm1(pre))
    xw2 = jnp.dot(h1, wdec_ref[...], preferred_element_type=jnp.float32)
    xws2_ref[:N, :] = xw2 * dinv
    xws2_ref[N:, :] = jnp.zeros((N_PAD - N, D), jnp.float32)


def _tc_post_body(q_ref, xws2_ref, dinv_ref, bdec_ref, out_ref):
    agg = q_ref[0, :N, :] + q_ref[1, :N, :] + xws2_ref[:N, :]
    out_ref[...] = dinv_ref[...] * agg + bdec_ref[...]


def kernel(z, edge_index, t, W_t1, b_t1, W_t2, b_t2, W_enc, b_enc, W_dec,
           b_dec):
    src = edge_index[0].astype(jnp.int32)
    dst = edge_index[1].astype(jnp.int32)
    padfill = jnp.full((EP - E,), N, dtype=jnp.int32)
    srcp = jnp.concatenate([src, padfill])
    dstp = jnp.concatenate([dst, padfill])

    ones_ch = jnp.ones((CH,), jnp.float32)
    zvec = jnp.zeros((ROWS_PER_TILE,), jnp.float32)
    zrows = jnp.zeros((CH, D), jnp.float32)

    degp = _sc_degree(dstp, ones_ch, zvec)

    xws1, dinv, temb = pl.pallas_call(
        _tc_pre_body,
        out_shape=(
            jax.ShapeDtypeStruct((N_PAD, D), jnp.float32),
            jax.ShapeDtypeStruct((N, 1), jnp.float32),
            jax.ShapeDtypeStruct((N, D), jnp.float32),
        ),
    )(degp, t.astype(jnp.int32).reshape(N, 1), z, W_t1, b_t1.reshape(1, D),
      W_t2, b_t2.reshape(1, D), W_enc)

    p = _sc_aggregate(xws1, srcp, dstp, zrows)

    xws2 = pl.pallas_call(
        _tc_mid_body,
        out_shape=jax.ShapeDtypeStruct((N_PAD, D), jnp.float32),
    )(p, xws1, dinv, temb, b_enc.reshape(1, D), W_dec)

    q = _sc_aggregate(xws2, srcp, dstp, zrows)

    out = pl.pallas_call(
        _tc_post_body,
        out_shape=jax.ShapeDtypeStruct((N, D), jnp.float32),
    )(q, xws2, dinv, b_dec.reshape(1, D))
    return out


# speedup vs baseline: 10.2295x; 10.2295x over previous
import functools
import jax
import jax.numpy as jnp
from jax import lax
from jax.experimental import pallas as pl
from jax.experimental.pallas import tpu as pltpu
from jax.experimental.pallas import tpu_sc as plsc

N = 10000
E = 320000
D = 128

NC = 2
NS = 16
NW = NC * NS

CH = 128
EP_PER_W = 10112
EP = NW * EP_PER_W
NCHUNK = EP_PER_W // CH
N_PAD = 10240
ROWS_PER_TILE = N_PAD // NS
NDUMP = ROWS_PER_TILE // CH

_mesh = plsc.VectorSubcoreMesh(core_axis_name="c", subcore_axis_name="s",
                               num_cores=NC, num_subcores=NS)


@functools.partial(
    pl.kernel,
    out_type=jax.ShapeDtypeStruct((NC, N_PAD), jnp.float32),
    mesh=_mesh,
    scratch_types=[
        pltpu.VMEM_SHARED((N_PAD,), jnp.float32),
        pltpu.VMEM((CH,), jnp.int32),
        pltpu.VMEM((CH,), jnp.float32),
        pltpu.VMEM((ROWS_PER_TILE,), jnp.float32),
        pltpu.SemaphoreType.DMA,
    ],
)
def _sc_degree(dst_hbm, ones_hbm, zvec_hbm, deg_out, acc_sh, idx_v, ones_v,
               zbuf_v, sem):
    c = lax.axis_index("c")
    s = lax.axis_index("s")
    wid = c * NS + s
    pltpu.sync_copy(zvec_hbm, zbuf_v)
    pltpu.sync_copy(zbuf_v, acc_sh.at[pl.ds(s * ROWS_PER_TILE, ROWS_PER_TILE)])
    pltpu.sync_copy(ones_hbm, ones_v)
    plsc.subcore_barrier()

    def body(j, carry):
        base = wid * EP_PER_W + j * CH
        pltpu.sync_copy(dst_hbm.at[pl.ds(base, CH)], idx_v)
        pltpu.sync_copy(ones_v, acc_sh.at[idx_v], add=True)
        return carry

    lax.fori_loop(0, NCHUNK, body, 0)
    plsc.subcore_barrier()
    r0 = s * ROWS_PER_TILE
    pltpu.sync_copy(acc_sh.at[pl.ds(r0, ROWS_PER_TILE)], zbuf_v)
    pltpu.sync_copy(zbuf_v, deg_out.at[c, pl.ds(r0, ROWS_PER_TILE)])


@functools.partial(
    pl.kernel,
    out_type=jax.ShapeDtypeStruct((NC, N_PAD, D), jnp.float32),
    mesh=_mesh,
    scratch_types=[
        pltpu.VMEM_SHARED((N_PAD, D), jnp.float32),
        pltpu.VMEM((CH,), jnp.int32),
        pltpu.VMEM((CH,), jnp.int32),
        pltpu.VMEM((CH, D), jnp.float32),
        pltpu.SemaphoreType.DMA,
    ],
)
def _sc_aggregate(xws_hbm, src_hbm, dst_hbm, zrows_hbm, agg_out, acc_sh,
                  idxs_v, idxd_v, rows_v, sem):
    c = lax.axis_index("c")
    s = lax.axis_index("s")
    wid = c * NS + s
    pltpu.sync_copy(zrows_hbm, rows_v)
    for k in range(NDUMP):
        pltpu.sync_copy(
            rows_v, acc_sh.at[pl.ds(s * ROWS_PER_TILE + k * CH, CH), :])
    plsc.subcore_barrier()

    def body(j, carry):
        base = wid * EP_PER_W + j * CH
        pltpu.sync_copy(src_hbm.at[pl.ds(base, CH)], idxs_v)
        pltpu.sync_copy(dst_hbm.at[pl.ds(base, CH)], idxd_v)
        pltpu.async_copy(xws_hbm.at[idxs_v], rows_v, sem).wait()
        pltpu.sync_copy(rows_v, acc_sh.at[idxd_v], add=True)
        return carry

    lax.fori_loop(0, NCHUNK, body, 0)
    plsc.subcore_barrier()
    for k in range(NDUMP):
        r0 = s * ROWS_PER_TILE + k * CH
        pltpu.sync_copy(acc_sh.at[pl.ds(r0, CH), :], rows_v)
        pltpu.sync_copy(rows_v, agg_out.at[c, pl.ds(r0, CH), :])


def _tc_pre_body(degp_ref, t_ref, z_ref, wt1_ref, bt1_ref, wt2_ref, bt2_ref,
                 wenc_ref, xws1_ref, dinv_ref, temb_ref):
    deg = degp_ref[0, :] + degp_ref[1, :] + 1.0
    dinv_all = lax.rsqrt(deg)
    dinv = dinv_all[:N].reshape(N, 1)
    dinv_ref[...] = dinv
    t_in = t_ref[...].astype(jnp.float32)
    h = t_in * wt1_ref[...] + bt1_ref[...]
    h = 0.5 * h * (1.0 + lax.erf(h * 0.7071067811865476))
    temb_ref[...] = jnp.dot(h, wt2_ref[...],
                            preferred_element_type=jnp.float32) + bt2_ref[...]
    xw1 = jnp.dot(z_ref[...], wenc_ref[...],
                  preferred_element_type=jnp.float32)
    xws1_ref[:N, :] = xw1 * dinv
    xws1_ref[N:, :] = jnp.zeros((N_PAD - N, D), jnp.float32)


def _tc_mid_body(p_ref, xws1_ref, dinv_ref, temb_ref, benc_ref, wdec_ref,
                 xws2_ref):
    dinv = dinv_ref[...]
    agg = p_ref[0, :N, :] + p_ref[1, :N, :] + xws1_ref[:N, :]
    pre = dinv * agg + benc_ref[...] + temb_ref[...]
    h1 = jnp.where(pre > 0, pre, jnp.exp(jnp.minimum(pre, 0.0)) - 1.0)
    xw2 = jnp.dot(h1, wdec_ref[...], preferred_element_type=jnp.float32)
    xws2_ref[:N, :] = xw2 * dinv
    xws2_ref[N:, :] = jnp.zeros((N_PAD - N, D), jnp.float32)


def _tc_post_body(q_ref, xws2_ref, dinv_ref, bdec_ref, out_ref):
    agg = q_ref[0, :N, :] + q_ref[1, :N, :] + xws2_ref[:N, :]
    out_ref[...] = dinv_ref[...] * agg + bdec_ref[...]


def kernel(z, edge_index, t, W_t1, b_t1, W_t2, b_t2, W_enc, b_enc, W_dec,
           b_dec):
    src = edge_index[0].astype(jnp.int32)
    dst = edge_index[1].astype(jnp.int32)
    padfill = jnp.full((EP - E,), N, dtype=jnp.int32)
    srcp = jnp.concatenate([src, padfill])
    dstp = jnp.concatenate([dst, padfill])

    ones_ch = jnp.ones((CH,), jnp.float32)
    zvec = jnp.zeros((ROWS_PER_TILE,), jnp.float32)
    zrows = jnp.zeros((CH, D), jnp.float32)

    degp = _sc_degree(dstp, ones_ch, zvec)

    xws1, dinv, temb = pl.pallas_call(
        _tc_pre_body,
        out_shape=(
            jax.ShapeDtypeStruct((N_PAD, D), jnp.float32),
            jax.ShapeDtypeStruct((N, 1), jnp.float32),
            jax.ShapeDtypeStruct((N, D), jnp.float32),
        ),
    )(degp, t.astype(jnp.int32).reshape(N, 1), z, W_t1, b_t1.reshape(1, D),
      W_t2, b_t2.reshape(1, D), W_enc)

    p = _sc_aggregate(xws1, srcp, dstp, zrows)

    xws2 = pl.pallas_call(
        _tc_mid_body,
        out_shape=jax.ShapeDtypeStruct((N_PAD, D), jnp.float32),
    )(p, xws1, dinv, temb, b_enc.reshape(1, D), W_dec)

    q = _sc_aggregate(xws2, srcp, dstp, zrows)

    out = pl.pallas_call(
        _tc_post_body,
        out_shape=jax.ShapeDtypeStruct((N, D), jnp.float32),
    )(q, xws2, dinv, b_dec.reshape(1, D))
    return out
